# Initial kernel scaffold; baseline (speedup 1.0000x reference)
import jax
import jax.numpy as jnp
from jax.experimental import pallas as pl


def kernel(inputs, batch_index):
    raise NotImplementedError("write your pallas kernel here")


# speedup vs baseline: 3.8071x; 3.8071x over previous
import functools

import jax
import jax.numpy as jnp
from jax import lax
from jax.experimental import pallas as pl
from jax.experimental.pallas import tpu as pltpu
from jax.experimental.pallas import tpu_sc as plsc

_B = 16
_M = 1024
_K = 16
_F = 256
_N = 4096

_NC = 2
_NS = 16
_NW = _NC * _NS
_RPW = (_B * _M) // _NW
_G = 8
_NCH = _RPW // _G
_IPC = _G * _K
_LANES = 16


def _body(tbl, idx, out, idxv, rows, outv, gsem0, gsem1, osem0, osem1):
    wid = lax.axis_index("s") * _NC + lax.axis_index("c")
    boff = (wid // (_NW // _B)) * _N

    pltpu.sync_copy(idx.at[wid], idxv)

    def add_off(i, _):
        for j in range(_IPC // _LANES):
            sl = (i, pl.ds(j * _LANES, _LANES))
            idxv[sl] = idxv[sl] + boff
        return 0

    lax.fori_loop(0, _NCH, add_off, 0)

    gsems = (gsem0, gsem1)
    osems = (osem0, osem1)

    def start_gather(c, buf):
        pltpu.async_copy(tbl.at[idxv.at[c]], rows.at[buf], gsems[buf])

    def wait_gather(c, buf):
        pltpu.make_async_copy(tbl.at[idxv.at[c]], rows.at[buf], gsems[buf]).wait()

    def wait_out(buf):
        pltpu.make_async_copy(outv.at[buf], out.at[pl.ds(0, _G)], osems[buf]).wait()

    def compute(c, buf):
        @pl.when(c >= 2)
        def _():
            wait_out(buf)

        def per_g(g, _):
            r0 = g * _K

            def per_d(d, __):
                s = pl.ds(d * _LANES, _LANES)
                v = [rows[buf, r0 + j, s] for j in range(_K)]
                while len(v) > 1:
                    v = [jnp.maximum(v[k], v[k + 1]) for k in range(0, len(v), 2)]
                outv[buf, g, s] = v[0]
                return 0

            return lax.fori_loop(0, _F // _LANES, per_d, 0)

        lax.fori_loop(0, _G, per_g, 0)

    def put_out(c, buf):
        base = wid * _RPW + c * _G
        pltpu.async_copy(outv.at[buf], out.at[pl.ds(base, _G)], osems[buf])

    start_gather(0, 0)

    def step(i, _):
        c0 = i * 2
        start_gather(c0 + 1, 1)
        wait_gather(c0, 0)
        compute(c0, 0)
        put_out(c0, 0)

        @pl.when(c0 + 2 < _NCH)
        def _():
            start_gather(c0 + 2, 0)

        wait_gather(c0 + 1, 1)
        compute(c0 + 1, 1)
        put_out(c0 + 1, 1)
        return 0

    lax.fori_loop(0, _NCH // 2, step, 0)
    wait_out(0)
    wait_out(1)


@jax.jit
def kernel(inputs, batch_index):
    tbl = inputs.reshape(_B * _N, _F)
    idx = batch_index.reshape(_NW, _NCH, _IPC)
    mesh = plsc.VectorSubcoreMesh(core_axis_name="c", subcore_axis_name="s")
    kern = pl.kernel(
        _body,
        mesh=mesh,
        out_type=jax.ShapeDtypeStruct((_B * _M, _F), jnp.float32),
        scratch_types=[
            pltpu.VMEM((_NCH, _IPC), jnp.int32),
            pltpu.VMEM((2, _IPC, _F), jnp.float32),
            pltpu.VMEM((2, _G, _F), jnp.float32),
            pltpu.SemaphoreType.DMA,
            pltpu.SemaphoreType.DMA,
            pltpu.SemaphoreType.DMA,
            pltpu.SemaphoreType.DMA,
        ],
    )
    out = kern(tbl, idx)
    return out.reshape(_B, _M, _F)
